# baseline (speedup 1.0000x reference)
import functools

import jax
import jax.numpy as jnp
from jax import lax
from jax.experimental import pallas as pl
from jax.experimental.pallas import tpu as pltpu
from jax.experimental.pallas import tpu_sc as plsc

VOCAB = 100000
D_EMB = 128
N_PROMPT = 20
BATCH = 4
SEQ_LEN = 2048

_ROWS = BATCH * SEQ_LEN
_NW = 32
_R_PER_W = _ROWS // _NW
_CHUNK = 64
_NCHUNK = _R_PER_W // _CHUNK
_W_PER_BATCH = _NW // BATCH


def _make_kernel():
    mesh = plsc.VectorSubcoreMesh(core_axis_name="c", subcore_axis_name="s")

    @functools.partial(
        pl.kernel,
        mesh=mesh,
        out_type=jax.ShapeDtypeStruct((_ROWS, D_EMB), jnp.float32),
        scratch_types=[
            pltpu.VMEM((BATCH, _R_PER_W), jnp.int32),
            pltpu.VMEM((_R_PER_W, D_EMB), jnp.float32),
            pltpu.VMEM((24, D_EMB), jnp.float32),
            pltpu.SemaphoreType.DMA,
        ]
        + [pltpu.SemaphoreType.DMA] * _NCHUNK,
    )
    def k(x_hbm, table_hbm, prompt_hbm, out_hbm, idx_v, rows_v, prompt_v,
          psem, *sems):
        wid = lax.axis_index("c") * 16 + lax.axis_index("s")
        base = wid * _R_PER_W
        b = wid // _W_PER_BATCH
        p0 = (wid % _W_PER_BATCH) * _R_PER_W
        is_head = wid % _W_PER_BATCH == 0
        prompt_cp = pltpu.async_copy(prompt_hbm, prompt_v.at[pl.ds(0, N_PROMPT)], psem)
        pltpu.sync_copy(x_hbm.at[:, pl.ds(p0, _R_PER_W)], idx_v)
        gathers = [
            pltpu.async_copy(
                table_hbm.at[idx_v.at[b, pl.ds(j * _CHUNK, _CHUNK)]],
                rows_v.at[pl.ds(j * _CHUNK, _CHUNK)],
                sems[j],
            )
            for j in range(_NCHUNK)
        ]
        writes = []
        for j in range(_NCHUNK):
            gathers[j].wait()
            writes.append(
                pltpu.async_copy(
                    rows_v.at[pl.ds(j * _CHUNK, _CHUNK)],
                    out_hbm.at[pl.ds(base + j * _CHUNK, _CHUNK)],
                    sems[j],
                )
            )
        @pl.when(is_head)
        def _():
            for r in range(N_PROMPT, 24):
                for c0 in range(0, D_EMB, 16):
                    prompt_v[r, pl.ds(c0, 16)] = rows_v[r, pl.ds(c0, 16)]

        writes[0].wait()

        @pl.when(is_head)
        def _():
            prompt_cp.wait()
            pltpu.async_copy(prompt_v, out_hbm.at[pl.ds(base, 24)], psem).wait()

        @pl.when(jnp.logical_not(is_head))
        def _():
            prompt_cp.wait()

        for w in writes[1:]:
            w.wait()

    return k


_kernel_call = _make_kernel()


def kernel(x, table, prompt):
    out = _kernel_call(x, table, prompt.reshape(N_PROMPT, D_EMB))
    return out.reshape(BATCH, SEQ_LEN, D_EMB)


# speedup vs baseline: 1.0069x; 1.0069x over previous
import functools

import jax
import jax.numpy as jnp
from jax import lax
from jax.experimental import pallas as pl
from jax.experimental.pallas import tpu as pltpu
from jax.experimental.pallas import tpu_sc as plsc

VOCAB = 100000
D_EMB = 128
N_PROMPT = 20
BATCH = 4
SEQ_LEN = 2048

_ROWS = BATCH * SEQ_LEN
_NW = 32
_R_PER_W = _ROWS // _NW
_CHUNK = 64
_NCHUNK = _R_PER_W // _CHUNK
_W_PER_BATCH = _NW // BATCH


def _make_kernel():
    mesh = plsc.VectorSubcoreMesh(core_axis_name="c", subcore_axis_name="s")

    @functools.partial(
        pl.kernel,
        mesh=mesh,
        out_type=jax.ShapeDtypeStruct((_ROWS, D_EMB), jnp.float32),
        scratch_types=[
            pltpu.VMEM((_R_PER_W,), jnp.int32),
            pltpu.VMEM((_R_PER_W, D_EMB), jnp.float32),
            pltpu.VMEM((24, D_EMB), jnp.float32),
            pltpu.SemaphoreType.DMA,
        ]
        + [pltpu.SemaphoreType.DMA] * _NCHUNK,
    )
    def k(x_hbm, table_hbm, prompt_hbm, out_hbm, idx_v, rows_v, prompt_v,
          psem, *sems):
        wid = lax.axis_index("c") * 16 + lax.axis_index("s")
        base = wid * _R_PER_W
        b = wid // _W_PER_BATCH
        p0 = (wid % _W_PER_BATCH) * _R_PER_W
        is_head = wid % _W_PER_BATCH == 0
        prompt_cp = pltpu.async_copy(
            prompt_hbm.at[0], prompt_v.at[pl.ds(0, N_PROMPT)], psem
        )
        pltpu.sync_copy(x_hbm.at[b, pl.ds(p0, _R_PER_W)], idx_v)
        gathers = [
            pltpu.async_copy(
                table_hbm.at[idx_v.at[pl.ds(j * _CHUNK, _CHUNK)]],
                rows_v.at[pl.ds(j * _CHUNK, _CHUNK)],
                sems[j],
            )
            for j in range(_NCHUNK)
        ]
        writes = []
        for j in range(_NCHUNK):
            gathers[j].wait()
            writes.append(
                pltpu.async_copy(
                    rows_v.at[pl.ds(j * _CHUNK, _CHUNK)],
                    out_hbm.at[pl.ds(base + j * _CHUNK, _CHUNK)],
                    sems[j],
                )
            )
        @pl.when(is_head)
        def _():
            for r in range(N_PROMPT, 24):
                for c0 in range(0, D_EMB, 16):
                    prompt_v[r, pl.ds(c0, 16)] = rows_v[r, pl.ds(c0, 16)]

        writes[0].wait()

        @pl.when(is_head)
        def _():
            prompt_cp.wait()
            pltpu.async_copy(prompt_v, out_hbm.at[pl.ds(base, 24)], psem).wait()

        @pl.when(jnp.logical_not(is_head))
        def _():
            prompt_cp.wait()

        for w in writes[1:]:
            w.wait()

    return k


_kernel_call = _make_kernel()


def kernel(x, table, prompt):
    out = _kernel_call(x, table, prompt)
    return out.reshape(BATCH, SEQ_LEN, D_EMB)


# speedup vs baseline: 1.0074x; 1.0005x over previous
import functools

import jax
import jax.numpy as jnp
from jax import lax
from jax.experimental import pallas as pl
from jax.experimental.pallas import tpu as pltpu
from jax.experimental.pallas import tpu_sc as plsc

VOCAB = 100000
D_EMB = 128
N_PROMPT = 20
BATCH = 4
SEQ_LEN = 2048

_ROWS = BATCH * SEQ_LEN
_NW = 32
_R_PER_W = _ROWS // _NW
_CHUNK = 32
_NCHUNK = _R_PER_W // _CHUNK
_W_PER_BATCH = _NW // BATCH


def _make_kernel():
    mesh = plsc.VectorSubcoreMesh(core_axis_name="c", subcore_axis_name="s")

    @functools.partial(
        pl.kernel,
        mesh=mesh,
        out_type=jax.ShapeDtypeStruct((_ROWS, D_EMB), jnp.float32),
        scratch_types=[
            pltpu.VMEM((_R_PER_W,), jnp.int32),
            pltpu.VMEM((_R_PER_W, D_EMB), jnp.float32),
            pltpu.VMEM((24, D_EMB), jnp.float32),
            pltpu.SemaphoreType.DMA,
        ]
        + [pltpu.SemaphoreType.DMA] * _NCHUNK,
    )
    def k(x_hbm, table_hbm, prompt_hbm, out_hbm, idx_v, rows_v, prompt_v,
          psem, *sems):
        wid = lax.axis_index("c") * 16 + lax.axis_index("s")
        base = wid * _R_PER_W
        b = wid // _W_PER_BATCH
        p0 = (wid % _W_PER_BATCH) * _R_PER_W
        is_head = wid % _W_PER_BATCH == 0
        prompt_cp = pltpu.async_copy(
            prompt_hbm.at[0], prompt_v.at[pl.ds(0, N_PROMPT)], psem
        )
        pltpu.sync_copy(x_hbm.at[b, pl.ds(p0, _R_PER_W)], idx_v)
        gathers = [
            pltpu.async_copy(
                table_hbm.at[idx_v.at[pl.ds(j * _CHUNK, _CHUNK)]],
                rows_v.at[pl.ds(j * _CHUNK, _CHUNK)],
                sems[j],
            )
            for j in range(_NCHUNK)
        ]
        writes = []
        for j in range(_NCHUNK):
            gathers[j].wait()
            writes.append(
                pltpu.async_copy(
                    rows_v.at[pl.ds(j * _CHUNK, _CHUNK)],
                    out_hbm.at[pl.ds(base + j * _CHUNK, _CHUNK)],
                    sems[j],
                )
            )
        @pl.when(is_head)
        def _():
            for r in range(N_PROMPT, 24):
                for c0 in range(0, D_EMB, 16):
                    prompt_v[r, pl.ds(c0, 16)] = rows_v[r, pl.ds(c0, 16)]

        writes[0].wait()

        @pl.when(is_head)
        def _():
            prompt_cp.wait()
            pltpu.async_copy(prompt_v, out_hbm.at[pl.ds(base, 24)], psem).wait()

        @pl.when(jnp.logical_not(is_head))
        def _():
            prompt_cp.wait()

        for w in writes[1:]:
            w.wait()

    return k


_kernel_call = _make_kernel()


def kernel(x, table, prompt):
    out = _kernel_call(x, table, prompt)
    return out.reshape(BATCH, SEQ_LEN, D_EMB)


# speedup vs baseline: 1.0190x; 1.0115x over previous
import functools

import jax
import jax.numpy as jnp
from jax import lax
from jax.experimental import pallas as pl
from jax.experimental.pallas import tpu as pltpu
from jax.experimental.pallas import tpu_sc as plsc

VOCAB = 100000
D_EMB = 128
N_PROMPT = 20
BATCH = 4
SEQ_LEN = 2048

_ROWS = BATCH * SEQ_LEN
_NW = 32
_R_PER_W = _ROWS // _NW
_CHUNK = 128
_NCHUNK = _R_PER_W // _CHUNK
_W_PER_BATCH = _NW // BATCH


def _make_kernel():
    mesh = plsc.VectorSubcoreMesh(core_axis_name="c", subcore_axis_name="s")

    @functools.partial(
        pl.kernel,
        mesh=mesh,
        out_type=jax.ShapeDtypeStruct((_ROWS, D_EMB), jnp.float32),
        scratch_types=[
            pltpu.VMEM((_R_PER_W,), jnp.int32),
            pltpu.VMEM((_R_PER_W, D_EMB), jnp.float32),
            pltpu.VMEM((24, D_EMB), jnp.float32),
            pltpu.SemaphoreType.DMA,
        ]
        + [pltpu.SemaphoreType.DMA] * _NCHUNK,
    )
    def k(x_hbm, table_hbm, prompt_hbm, out_hbm, idx_v, rows_v, prompt_v,
          psem, *sems):
        wid = lax.axis_index("c") * 16 + lax.axis_index("s")
        base = wid * _R_PER_W
        b = wid // _W_PER_BATCH
        p0 = (wid % _W_PER_BATCH) * _R_PER_W
        is_head = wid % _W_PER_BATCH == 0
        prompt_cp = pltpu.async_copy(
            prompt_hbm.at[0], prompt_v.at[pl.ds(0, N_PROMPT)], psem
        )
        pltpu.sync_copy(x_hbm.at[b, pl.ds(p0, _R_PER_W)], idx_v)
        gathers = [
            pltpu.async_copy(
                table_hbm.at[idx_v.at[pl.ds(j * _CHUNK, _CHUNK)]],
                rows_v.at[pl.ds(j * _CHUNK, _CHUNK)],
                sems[j],
            )
            for j in range(_NCHUNK)
        ]
        writes = []
        for j in range(_NCHUNK):
            gathers[j].wait()
            writes.append(
                pltpu.async_copy(
                    rows_v.at[pl.ds(j * _CHUNK, _CHUNK)],
                    out_hbm.at[pl.ds(base + j * _CHUNK, _CHUNK)],
                    sems[j],
                )
            )
        @pl.when(is_head)
        def _():
            for r in range(N_PROMPT, 24):
                for c0 in range(0, D_EMB, 16):
                    prompt_v[r, pl.ds(c0, 16)] = rows_v[r, pl.ds(c0, 16)]

        writes[0].wait()

        @pl.when(is_head)
        def _():
            prompt_cp.wait()
            pltpu.async_copy(prompt_v, out_hbm.at[pl.ds(base, 24)], psem).wait()

        @pl.when(jnp.logical_not(is_head))
        def _():
            prompt_cp.wait()

        for w in writes[1:]:
            w.wait()

    return k


_kernel_call = _make_kernel()


def kernel(x, table, prompt):
    out = _kernel_call(x, table, prompt)
    return out.reshape(BATCH, SEQ_LEN, D_EMB)


# speedup vs baseline: 1.0244x; 1.0053x over previous
import functools

import jax
import jax.numpy as jnp
from jax import lax
from jax.experimental import pallas as pl
from jax.experimental.pallas import tpu as pltpu
from jax.experimental.pallas import tpu_sc as plsc

VOCAB = 100000
D_EMB = 128
N_PROMPT = 20
BATCH = 4
SEQ_LEN = 2048

_ROWS = BATCH * SEQ_LEN
_R0 = 240
_R1 = 272
_CHUNKS0 = (128, 112)
_CHUNKS1 = (128, 128, 16)
_IDXWIN = 384
_NSEM = 3


def _make_kernel():
    mesh = plsc.VectorSubcoreMesh(core_axis_name="c", subcore_axis_name="s")

    @functools.partial(
        pl.kernel,
        mesh=mesh,
        out_type=jax.ShapeDtypeStruct((_ROWS, D_EMB), jnp.float32),
        scratch_types=[
            pltpu.VMEM((_IDXWIN,), jnp.int32),
            pltpu.VMEM((_R1, D_EMB), jnp.float32),
            pltpu.VMEM((24, D_EMB), jnp.float32),
            pltpu.SemaphoreType.DMA,
        ]
        + [pltpu.SemaphoreType.DMA] * _NSEM,
    )
    def k(x_hbm, table_hbm, prompt_hbm, out_hbm, idx_v, rows_v, prompt_v,
          psem, *sems):
        c = lax.axis_index("c")
        s = lax.axis_index("s")
        b = s // 4
        kk = s % 4
        p0 = jnp.where(c == 0, _R0 * kk, 960 + _R1 * kk)
        p0 = pl.multiple_of(p0, 8)
        base = pl.multiple_of(b * SEQ_LEN + p0, 8)
        is_head = jnp.logical_and(c == 0, kk == 0)
        prompt_cp = pltpu.async_copy(
            prompt_hbm.at[0], prompt_v.at[pl.ds(0, N_PROMPT)], psem
        )
        w0 = pl.multiple_of((p0 // 128) * 128, 128)
        off = pl.multiple_of(p0 - w0, 8)
        pltpu.sync_copy(x_hbm.at[b, pl.ds(w0, _IDXWIN)], idx_v)

        def run(chunks):
            gathers = []
            cum = 0
            for j, sz in enumerate(chunks):
                gathers.append(
                    pltpu.async_copy(
                        table_hbm.at[
                            idx_v.at[pl.ds(pl.multiple_of(off + cum, 8), sz)]
                        ],
                        rows_v.at[pl.ds(cum, sz)],
                        sems[j],
                    )
                )
                cum += sz
            writes = []
            cum = 0
            for j, sz in enumerate(chunks):
                gathers[j].wait()
                writes.append(
                    pltpu.async_copy(
                        rows_v.at[pl.ds(cum, sz)],
                        out_hbm.at[pl.ds(pl.multiple_of(base + cum, 8), sz)],
                        sems[j],
                    )
                )
                cum += sz
            return writes

        @pl.when(c == 0)
        def _():
            writes = run(_CHUNKS0)
            @pl.when(is_head)
            def _():
                for r in range(N_PROMPT, 24):
                    for c0 in range(0, D_EMB, 16):
                        prompt_v[r, pl.ds(c0, 16)] = rows_v[r, pl.ds(c0, 16)]

            writes[0].wait()

            @pl.when(is_head)
            def _():
                prompt_cp.wait()
                pltpu.async_copy(
                    prompt_v, out_hbm.at[pl.ds(base, 24)], psem
                ).wait()

            @pl.when(jnp.logical_not(is_head))
            def _():
                prompt_cp.wait()

            for w in writes[1:]:
                w.wait()

        @pl.when(c == 1)
        def _():
            writes = run(_CHUNKS1)
            prompt_cp.wait()
            for w in writes:
                w.wait()

    return k


_kernel_call = _make_kernel()


def kernel(x, table, prompt):
    out = _kernel_call(x, table, prompt)
    return out.reshape(BATCH, SEQ_LEN, D_EMB)


# speedup vs baseline: 1.0251x; 1.0007x over previous
import functools

import jax
import jax.numpy as jnp
from jax import lax
from jax.experimental import pallas as pl
from jax.experimental.pallas import tpu as pltpu
from jax.experimental.pallas import tpu_sc as plsc

VOCAB = 100000
D_EMB = 128
N_PROMPT = 20
BATCH = 4
SEQ_LEN = 2048

_ROWS = BATCH * SEQ_LEN
_R0 = 224
_R1 = 288
_CHUNKS0 = (128, 96)
_CHUNKS1 = (128, 128, 32)
_IDXWIN = 384
_NSEM = 3


def _make_kernel():
    mesh = plsc.VectorSubcoreMesh(core_axis_name="c", subcore_axis_name="s")

    @functools.partial(
        pl.kernel,
        mesh=mesh,
        out_type=jax.ShapeDtypeStruct((_ROWS, D_EMB), jnp.float32),
        scratch_types=[
            pltpu.VMEM((_IDXWIN,), jnp.int32),
            pltpu.VMEM((_R1, D_EMB), jnp.float32),
            pltpu.VMEM((24, D_EMB), jnp.float32),
            pltpu.SemaphoreType.DMA,
        ]
        + [pltpu.SemaphoreType.DMA] * _NSEM,
    )
    def k(x_hbm, table_hbm, prompt_hbm, out_hbm, idx_v, rows_v, prompt_v,
          psem, *sems):
        c = lax.axis_index("c")
        s = lax.axis_index("s")
        b = s // 4
        kk = s % 4
        p0 = jnp.where(c == 0, _R0 * kk, 896 + _R1 * kk)
        p0 = pl.multiple_of(p0, 8)
        base = pl.multiple_of(b * SEQ_LEN + p0, 8)
        is_head = jnp.logical_and(c == 0, kk == 0)
        prompt_cp = pltpu.async_copy(
            prompt_hbm.at[0], prompt_v.at[pl.ds(0, N_PROMPT)], psem
        )
        w0 = pl.multiple_of((p0 // 128) * 128, 128)
        off = pl.multiple_of(p0 - w0, 8)
        pltpu.sync_copy(x_hbm.at[b, pl.ds(w0, _IDXWIN)], idx_v)

        def run(chunks):
            gathers = []
            cum = 0
            for j, sz in enumerate(chunks):
                gathers.append(
                    pltpu.async_copy(
                        table_hbm.at[
                            idx_v.at[pl.ds(pl.multiple_of(off + cum, 8), sz)]
                        ],
                        rows_v.at[pl.ds(cum, sz)],
                        sems[j],
                    )
                )
                cum += sz
            writes = []
            cum = 0
            for j, sz in enumerate(chunks):
                gathers[j].wait()
                writes.append(
                    pltpu.async_copy(
                        rows_v.at[pl.ds(cum, sz)],
                        out_hbm.at[pl.ds(pl.multiple_of(base + cum, 8), sz)],
                        sems[j],
                    )
                )
                cum += sz
            return writes

        @pl.when(c == 0)
        def _():
            writes = run(_CHUNKS0)
            @pl.when(is_head)
            def _():
                for r in range(N_PROMPT, 24):
                    for c0 in range(0, D_EMB, 16):
                        prompt_v[r, pl.ds(c0, 16)] = rows_v[r, pl.ds(c0, 16)]

            writes[0].wait()

            @pl.when(is_head)
            def _():
                prompt_cp.wait()
                pltpu.async_copy(
                    prompt_v, out_hbm.at[pl.ds(base, 24)], psem
                ).wait()

            @pl.when(jnp.logical_not(is_head))
            def _():
                prompt_cp.wait()

            for w in writes[1:]:
                w.wait()

        @pl.when(c == 1)
        def _():
            writes = run(_CHUNKS1)
            prompt_cp.wait()
            for w in writes:
                w.wait()

    return k


_kernel_call = _make_kernel()


def kernel(x, table, prompt):
    out = _kernel_call(x, table, prompt)
    return out.reshape(BATCH, SEQ_LEN, D_EMB)


# speedup vs baseline: 1.0326x; 1.0073x over previous
import functools

import jax
import jax.numpy as jnp
from jax import lax
from jax.experimental import pallas as pl
from jax.experimental.pallas import tpu as pltpu
from jax.experimental.pallas import tpu_sc as plsc

VOCAB = 100000
D_EMB = 128
N_PROMPT = 20
BATCH = 4
SEQ_LEN = 2048

_ROWS = BATCH * SEQ_LEN
_R0 = 224
_R1 = 288
_CHUNKS0 = (128, 96)
_CHUNKS1 = (128, 128, 32)
_IDXWIN = 384
_NSEM = 3


def _make_kernel():
    mesh = plsc.VectorSubcoreMesh(core_axis_name="c", subcore_axis_name="s")

    @functools.partial(
        pl.kernel,
        mesh=mesh,
        out_type=jax.ShapeDtypeStruct((_ROWS, D_EMB), jnp.float32),
        scratch_types=[
            pltpu.VMEM((_IDXWIN,), jnp.int32),
            pltpu.VMEM((_R1, D_EMB), jnp.float32),
            pltpu.VMEM((24, D_EMB), jnp.float32),
            pltpu.SemaphoreType.DMA,
        ]
        + [pltpu.SemaphoreType.DMA] * _NSEM,
    )
    def k(x_hbm, table_hbm, prompt_hbm, out_hbm, idx_v, rows_v, prompt_v,
          psem, *sems):
        c = lax.axis_index("c")
        s = lax.axis_index("s")
        b = s // 4
        kk = s % 4
        p0 = jnp.where(c == 0, _R0 * kk, 896 + _R1 * kk)
        p0 = pl.multiple_of(p0, 8)
        base = pl.multiple_of(b * SEQ_LEN + p0, 8)
        is_head = jnp.logical_and(c == 0, kk == 0)
        prompt_cp = pltpu.async_copy(
            prompt_hbm.at[0], prompt_v.at[pl.ds(0, N_PROMPT)], psem
        )
        w0 = pl.multiple_of((p0 // 128) * 128, 128)
        off = pl.multiple_of(p0 - w0, 8)
        pltpu.sync_copy(x_hbm.at[b, pl.ds(w0, 256)], idx_v.at[pl.ds(0, 256)])

        def run(chunks):
            gathers = []
            cum = 0
            for j, sz in enumerate(chunks):
                gathers.append(
                    pltpu.async_copy(
                        table_hbm.at[
                            idx_v.at[pl.ds(pl.multiple_of(off + cum, 8), sz)]
                        ],
                        rows_v.at[pl.ds(cum, sz)],
                        sems[j],
                    )
                )
                cum += sz
                if j == 0:
                    pltpu.sync_copy(
                        x_hbm.at[b, pl.ds(pl.multiple_of(w0 + 256, 128), 128)],
                        idx_v.at[pl.ds(256, 128)],
                    )
            writes = []
            cum = 0
            for j, sz in enumerate(chunks):
                gathers[j].wait()
                writes.append(
                    pltpu.async_copy(
                        rows_v.at[pl.ds(cum, sz)],
                        out_hbm.at[pl.ds(pl.multiple_of(base + cum, 8), sz)],
                        sems[j],
                    )
                )
                cum += sz
            return writes

        @pl.when(c == 0)
        def _():
            writes = run(_CHUNKS0)
            @pl.when(is_head)
            def _():
                for r in range(N_PROMPT, 24):
                    for c0 in range(0, D_EMB, 16):
                        prompt_v[r, pl.ds(c0, 16)] = rows_v[r, pl.ds(c0, 16)]

            writes[0].wait()

            @pl.when(is_head)
            def _():
                prompt_cp.wait()
                pltpu.async_copy(
                    prompt_v, out_hbm.at[pl.ds(base, 24)], psem
                ).wait()

            @pl.when(jnp.logical_not(is_head))
            def _():
                prompt_cp.wait()

            for w in writes[1:]:
                w.wait()

        @pl.when(c == 1)
        def _():
            writes = run(_CHUNKS1)
            prompt_cp.wait()
            for w in writes:
                w.wait()

    return k


_kernel_call = _make_kernel()


def kernel(x, table, prompt):
    out = _kernel_call(x, table, prompt)
    return out.reshape(BATCH, SEQ_LEN, D_EMB)
